# Initial kernel scaffold; baseline (speedup 1.0000x reference)
import jax
import jax.numpy as jnp
from jax.experimental import pallas as pl


def kernel(x, edge_idx, W, b, gamma, beta):
    raise NotImplementedError("write your pallas kernel here")


# speedup vs baseline: 14.6123x; 14.6123x over previous
import functools

import jax
import jax.numpy as jnp
from jax import lax
from jax.experimental import pallas as pl
from jax.experimental.pallas import tpu as pltpu
from jax.experimental.pallas import tpu_sc as plsc

N = 10000
D = 128
E = 320000
ALPHA = 0.2

NC = 2
NS = 16
NW = NC * NS

CH = 79
E_PER_TILE = CH * 128
E_PAD = NW * E_PER_TILE
N_PAD = 10112
STRIPE = N_PAD // NS

_MESH = plsc.VectorSubcoreMesh(
    core_axis_name="c", subcore_axis_name="s", num_cores=NC, num_subcores=NS
)


DW = 16


def _deg_body(dst_hbm, deg_out, dst_v, ones_buf, acc):
    c = lax.axis_index("c")
    s = lax.axis_index("s")
    w = c * NS + s
    zeros16 = jnp.zeros((16,), jnp.float32)

    def zero(r, _):
        ones_buf[r, pl.ds(0, 16)] = zeros16
        return 0

    lax.fori_loop(0, 128, zero, 0)
    base = s * STRIPE
    for k in range(STRIPE // 128):
        pltpu.sync_copy(ones_buf, acc.at[pl.ds(base + k * 128, 128)])
    rem = STRIPE % 128
    pltpu.sync_copy(ones_buf.at[pl.ds(0, rem)], acc.at[pl.ds(base + STRIPE - rem, rem)])
    ones16 = jnp.ones((16,), jnp.float32)

    def fill(r, _):
        ones_buf[r, pl.ds(0, 16)] = ones16
        return 0

    lax.fori_loop(0, 128, fill, 0)
    pltpu.sync_copy(dst_hbm.at[w], dst_v)
    plsc.subcore_barrier()

    def step(j, _):
        pltpu.sync_copy(ones_buf, acc.at[dst_v.at[j]], add=True)
        return 0

    lax.fori_loop(0, CH, step, 0)
    plsc.subcore_barrier()
    pltpu.sync_copy(acc.at[pl.ds(base, STRIPE)], deg_out.at[c, pl.ds(base, STRIPE)])


_deg_call = pl.kernel(
    _deg_body,
    out_type=jax.ShapeDtypeStruct((NC, N_PAD, DW), jnp.float32),
    mesh=_MESH,
    scratch_types=[
        pltpu.VMEM((CH, 128), jnp.int32),
        pltpu.VMEM((128, DW), jnp.float32),
        pltpu.VMEM_SHARED((N_PAD, DW), jnp.float32),
    ],
)


def _mm_body(x_ref, w_ref, deg_ref, g_ref, dinv_ref):
    i = pl.program_id(0)
    deg = (deg_ref[0] + deg_ref[1])[:, 0:1] + 1.0
    dv = lax.rsqrt(deg)
    row = i * 128 + lax.broadcasted_iota(jnp.int32, (128, 1), 0)
    dv = jnp.where(row < N, dv, 0.0)
    h = jnp.dot(x_ref[...], w_ref[...], preferred_element_type=jnp.float32)
    g_ref[...] = dv * h
    dinv_ref[...] = dv


def _mm_call(x, W, deg):
    return pl.pallas_call(
        _mm_body,
        grid=(N_PAD // 128,),
        in_specs=[
            pl.BlockSpec((128, D), lambda i: (i, 0)),
            pl.BlockSpec((D, D), lambda i: (0, 0)),
            pl.BlockSpec((NC, 128, DW), lambda i: (0, i, 0)),
        ],
        out_specs=[
            pl.BlockSpec((128, D), lambda i: (i, 0)),
            pl.BlockSpec((128, 1), lambda i: (i, 0)),
        ],
        out_shape=[
            jax.ShapeDtypeStruct((N_PAD, D), jnp.float32),
            jax.ShapeDtypeStruct((N_PAD, 1), jnp.float32),
        ],
    )(x, W, deg)


def _agg_body(g_hbm, src_hbm, dst_hbm, out_hbm, src_v, dst_v, row_buf, acc, sem):
    c = lax.axis_index("c")
    s = lax.axis_index("s")
    w = c * NS + s
    zeros16 = jnp.zeros((16,), jnp.float32)

    def zero(r, _):
        for cc in range(8):
            row_buf[r, pl.ds(cc * 16, 16)] = zeros16
        return 0

    lax.fori_loop(0, 128, zero, 0)
    base = s * STRIPE
    for k in range(STRIPE // 128):
        pltpu.sync_copy(row_buf, acc.at[pl.ds(base + k * 128, 128)])
    rem = STRIPE % 128
    pltpu.sync_copy(
        row_buf.at[pl.ds(0, rem)], acc.at[pl.ds(base + STRIPE - rem, rem)]
    )
    pltpu.sync_copy(src_hbm.at[w], src_v)
    pltpu.sync_copy(dst_hbm.at[w], dst_v)
    plsc.subcore_barrier()

    def step(j, _):
        pltpu.async_copy(g_hbm.at[src_v.at[j]], row_buf, sem).wait()
        pltpu.sync_copy(row_buf, acc.at[dst_v.at[j]], add=True)
        return 0

    lax.fori_loop(0, CH, step, 0)
    plsc.subcore_barrier()
    pltpu.sync_copy(acc.at[pl.ds(base, STRIPE)], out_hbm.at[c, pl.ds(base, STRIPE)])


_agg_call = pl.kernel(
    _agg_body,
    out_type=jax.ShapeDtypeStruct((NC, N_PAD, D), jnp.float32),
    mesh=_MESH,
    scratch_types=[
        pltpu.VMEM((CH, 128), jnp.int32),
        pltpu.VMEM((CH, 128), jnp.int32),
        pltpu.VMEM((128, D), jnp.float32),
        pltpu.VMEM_SHARED((N_PAD, D), jnp.float32),
        pltpu.SemaphoreType.DMA,
    ],
)


def _stats_body(p_ref, g_ref, dinv_ref, b_ref, t_ref, st_ref):
    i = pl.program_id(0)
    t = dinv_ref[...] * (p_ref[0] + p_ref[1] + g_ref[...]) + b_ref[...]
    t_ref[...] = t
    row = i * 128 + lax.broadcasted_iota(jnp.int32, (128, 1), 0)
    tm = jnp.where(row < N, t, 0.0)

    @pl.when(i == 0)
    def _():
        st_ref[...] = jnp.zeros_like(st_ref)

    s1 = jnp.sum(tm, axis=0, keepdims=True)
    s2 = jnp.sum(tm * tm, axis=0, keepdims=True)
    st_ref[...] = st_ref[...] + jnp.concatenate([s1, s2], axis=0)


def _stats_call(p, g, dinv, b):
    return pl.pallas_call(
        _stats_body,
        grid=(N_PAD // 128,),
        in_specs=[
            pl.BlockSpec((NC, 128, D), lambda i: (0, i, 0)),
            pl.BlockSpec((128, D), lambda i: (i, 0)),
            pl.BlockSpec((128, 1), lambda i: (i, 0)),
            pl.BlockSpec((1, D), lambda i: (0, 0)),
        ],
        out_specs=[
            pl.BlockSpec((128, D), lambda i: (i, 0)),
            pl.BlockSpec((2, D), lambda i: (0, 0)),
        ],
        out_shape=[
            jax.ShapeDtypeStruct((N_PAD, D), jnp.float32),
            jax.ShapeDtypeStruct((2, D), jnp.float32),
        ],
    )(p, g, dinv, b)


def _apply_body(t_ref, st_ref, gamma_ref, beta_ref, y_ref):
    st = st_ref[...]
    mean = st[0:1, :] * (1.0 / N)
    var = st[1:2, :] * (1.0 / N) - mean * mean
    inv = lax.rsqrt(var + 1e-5)
    y = (t_ref[...] - mean) * inv * gamma_ref[...] + beta_ref[...]
    y_ref[...] = jnp.where(y > 0, y, ALPHA * y)


def _apply_call(t, st, gamma, beta):
    return pl.pallas_call(
        _apply_body,
        grid=(N_PAD // 128,),
        in_specs=[
            pl.BlockSpec((128, D), lambda i: (i, 0)),
            pl.BlockSpec((2, D), lambda i: (0, 0)),
            pl.BlockSpec((1, D), lambda i: (0, 0)),
            pl.BlockSpec((1, D), lambda i: (0, 0)),
        ],
        out_specs=pl.BlockSpec((128, D), lambda i: (i, 0)),
        out_shape=jax.ShapeDtypeStruct((N_PAD, D), jnp.float32),
    )(t, st, gamma, beta)


@jax.jit
def kernel(x, edge_idx, W, b, gamma, beta):
    src = edge_idx[0].astype(jnp.int32)
    dst = edge_idx[1].astype(jnp.int32)
    pad = jnp.full((E_PAD - E,), N, jnp.int32)
    src_p = jnp.concatenate([src, pad]).reshape(NW, CH, 128)
    dst_p = jnp.concatenate([dst, pad]).reshape(NW, CH, 128)

    deg = _deg_call(dst_p)
    g, dinv = _mm_call(x, W, deg)
    p = _agg_call(g, src_p, dst_p)
    t, st = _stats_call(p, g, dinv, b.reshape(1, D))
    y = _apply_call(t, st, gamma.reshape(1, D), beta.reshape(1, D))
    return y[:N]


# speedup vs baseline: 15.0158x; 1.0276x over previous
import functools

import jax
import jax.numpy as jnp
from jax import lax
from jax.experimental import pallas as pl
from jax.experimental.pallas import tpu as pltpu
from jax.experimental.pallas import tpu_sc as plsc

N = 10000
D = 128
E = 320000
ALPHA = 0.2

NC = 2
NS = 16
NW = NC * NS

CH = 80
E_PER_TILE = CH * 128
E_PAD = NW * E_PER_TILE
N_CHUNK = E_PAD // 128
CH0 = 104
CH1 = (N_CHUNK - NS * CH0) // NS
C0TOT = NS * CH0
N_PAD = 10112
STRIPE = N_PAD // NS

_MESH = plsc.VectorSubcoreMesh(
    core_axis_name="c", subcore_axis_name="s", num_cores=NC, num_subcores=NS
)


DW = 16


def _deg_body(dst_hbm, deg_out, dst_v, ones_buf, acc):
    c = lax.axis_index("c")
    s = lax.axis_index("s")
    w = c * NS + s
    zeros16 = jnp.zeros((16,), jnp.float32)

    def zero(r, _):
        ones_buf[r, pl.ds(0, 16)] = zeros16
        return 0

    lax.fori_loop(0, 128, zero, 0)
    base = s * STRIPE
    for k in range(STRIPE // 128):
        pltpu.sync_copy(ones_buf, acc.at[pl.ds(base + k * 128, 128)])
    rem = STRIPE % 128
    pltpu.sync_copy(ones_buf.at[pl.ds(0, rem)], acc.at[pl.ds(base + STRIPE - rem, rem)])
    ones16 = jnp.ones((16,), jnp.float32)

    def fill(r, _):
        ones_buf[r, pl.ds(0, 16)] = ones16
        return 0

    lax.fori_loop(0, 128, fill, 0)
    pltpu.sync_copy(dst_hbm.at[w], dst_v)
    plsc.subcore_barrier()

    def step(j, _):
        pltpu.sync_copy(ones_buf, acc.at[dst_v.at[j]], add=True)
        return 0

    lax.fori_loop(0, CH, step, 0)
    plsc.subcore_barrier()
    pltpu.sync_copy(acc.at[pl.ds(base, STRIPE)], deg_out.at[c, pl.ds(base, STRIPE)])


_deg_call = pl.kernel(
    _deg_body,
    out_type=jax.ShapeDtypeStruct((NC, N_PAD, DW), jnp.float32),
    mesh=_MESH,
    scratch_types=[
        pltpu.VMEM((CH, 128), jnp.int32),
        pltpu.VMEM((128, DW), jnp.float32),
        pltpu.VMEM_SHARED((N_PAD, DW), jnp.float32),
    ],
)


MMB = N_PAD // 8


def _mm_body(x_ref, w_ref, deg_ref, g_ref, dinv_ref):
    i = pl.program_id(0)
    deg = (deg_ref[0] + deg_ref[1])[:, 0:1] + 1.0
    dv = lax.rsqrt(deg)
    row = i * MMB + lax.broadcasted_iota(jnp.int32, (MMB, 1), 0)
    dv = jnp.where(row < N, dv, 0.0)
    h = jnp.dot(x_ref[...], w_ref[...], preferred_element_type=jnp.float32)
    g_ref[...] = dv * h
    dinv_ref[...] = dv


def _mm_call(x, W, deg):
    return pl.pallas_call(
        _mm_body,
        grid=(N_PAD // MMB,),
        in_specs=[
            pl.BlockSpec((MMB, D), lambda i: (i, 0)),
            pl.BlockSpec((D, D), lambda i: (0, 0)),
            pl.BlockSpec((NC, MMB, DW), lambda i: (0, i, 0)),
        ],
        out_specs=[
            pl.BlockSpec((MMB, D), lambda i: (i, 0)),
            pl.BlockSpec((MMB, 1), lambda i: (i, 0)),
        ],
        out_shape=[
            jax.ShapeDtypeStruct((N_PAD, D), jnp.float32),
            jax.ShapeDtypeStruct((N_PAD, 1), jnp.float32),
        ],
    )(x, W, deg)


WIN = 8
NWIN0 = CH0 // WIN
NWIN1 = CH1 // WIN


def _agg_body(g_hbm, src_hbm, dst_hbm, out_hbm,
              sw0, dw0, sw1, dw1, buf0, buf1, acc, sem_g, sem_w):
    c = lax.axis_index("c")
    s = lax.axis_index("s")
    zeros16 = jnp.zeros((16,), jnp.float32)

    def zero(r, _):
        for cc in range(8):
            buf0[r, pl.ds(cc * 16, 16)] = zeros16
        return 0

    lax.fori_loop(0, 128, zero, 0)
    base = s * STRIPE
    for k in range(STRIPE // 128):
        pltpu.sync_copy(buf0, acc.at[pl.ds(base + k * 128, 128)])
    rem = STRIPE % 128
    pltpu.sync_copy(buf0.at[pl.ds(0, rem)], acc.at[pl.ds(base + STRIPE - rem, rem)])

    start = jnp.where(c == 0, s * CH0, C0TOT + s * CH1)
    nwin = jnp.where(c == 0, NWIN0, NWIN1)
    pltpu.sync_copy(src_hbm.at[pl.ds(pl.multiple_of(start, 8), WIN)], sw0)
    pltpu.sync_copy(dst_hbm.at[pl.ds(pl.multiple_of(start, 8), WIN)], dw0)
    plsc.subcore_barrier()
    pltpu.async_copy(g_hbm.at[sw0.at[pl.multiple_of(jnp.int32(0), 8)]], buf0, sem_g)

    def window_body(wi, w_cur_s, w_cur_d, w_next_s, w_next_d):
        @pl.when(wi + 1 < nwin)
        def _():
            off = pl.multiple_of(start + (wi + 1) * WIN, 8)
            pltpu.async_copy(src_hbm.at[pl.ds(off, WIN)], w_next_s, sem_w)
            pltpu.async_copy(dst_hbm.at[pl.ds(off, WIN)], w_next_d, sem_w)

        def pair(j2p, _):
            a = j2p * 2
            pltpu.make_async_copy(g_hbm.at[w_cur_s.at[a]], buf0, sem_g).wait()
            pltpu.async_copy(g_hbm.at[w_cur_s.at[a + 1]], buf1, sem_g)
            pltpu.sync_copy(buf0, acc.at[w_cur_d.at[a]], add=True)
            pltpu.make_async_copy(g_hbm.at[w_cur_s.at[a + 1]], buf1, sem_g).wait()

            @pl.when(j2p + 1 < WIN // 2)
            def _():
                pltpu.async_copy(g_hbm.at[w_cur_s.at[a + 2]], buf0, sem_g)

            @pl.when(j2p + 1 == WIN // 2)
            def _():
                @pl.when(wi + 1 < nwin)
                def _():
                    pltpu.make_async_copy(src_hbm.at[pl.ds(0, WIN)], w_next_s, sem_w).wait()
                    pltpu.make_async_copy(dst_hbm.at[pl.ds(0, WIN)], w_next_d, sem_w).wait()
                    zero = pl.multiple_of(jnp.int32(0), 8)
                    pltpu.async_copy(g_hbm.at[w_next_s.at[zero]], buf0, sem_g)

            pltpu.sync_copy(buf1, acc.at[w_cur_d.at[a + 1]], add=True)
            return 0

        lax.fori_loop(0, WIN // 2, pair, 0)

    def outer(wi, _):
        @pl.when(wi < nwin)
        def _():
            @pl.when(lax.rem(wi, 2) == 0)
            def _():
                window_body(wi, sw0, dw0, sw1, dw1)

            @pl.when(lax.rem(wi, 2) == 1)
            def _():
                window_body(wi, sw1, dw1, sw0, dw0)

        return 0

    lax.fori_loop(0, NWIN0, outer, 0)
    plsc.subcore_barrier()
    pltpu.sync_copy(acc.at[pl.ds(base, STRIPE)], out_hbm.at[c, pl.ds(base, STRIPE)])


_agg_call = pl.kernel(
    _agg_body,
    out_type=jax.ShapeDtypeStruct((NC, N_PAD, D), jnp.float32),
    mesh=_MESH,
    scratch_types=[
        pltpu.VMEM((WIN, 128), jnp.int32),
        pltpu.VMEM((WIN, 128), jnp.int32),
        pltpu.VMEM((WIN, 128), jnp.int32),
        pltpu.VMEM((WIN, 128), jnp.int32),
        pltpu.VMEM((128, D), jnp.float32),
        pltpu.VMEM((128, D), jnp.float32),
        pltpu.VMEM_SHARED((N_PAD, D), jnp.float32),
        pltpu.SemaphoreType.DMA,
        pltpu.SemaphoreType.DMA,
    ],
)


def _bn_body(p_ref, g_ref, dinv_ref, b_ref, gamma_ref, beta_ref, y_ref):
    t = dinv_ref[...] * (p_ref[0] + p_ref[1] + g_ref[...]) + b_ref[...]
    row = lax.broadcasted_iota(jnp.int32, (N_PAD, 1), 0)
    tm = jnp.where(row < N, t, 0.0)
    s1 = jnp.sum(tm, axis=0, keepdims=True)
    s2 = jnp.sum(tm * tm, axis=0, keepdims=True)
    mean = s1 * (1.0 / N)
    var = s2 * (1.0 / N) - mean * mean
    inv = lax.rsqrt(var + 1e-5)
    y = (t - mean) * inv * gamma_ref[...] + beta_ref[...]
    y_ref[...] = jnp.where(y > 0, y, ALPHA * y)


def _bn_call(p, g, dinv, b, gamma, beta):
    return pl.pallas_call(
        _bn_body,
        out_shape=jax.ShapeDtypeStruct((N_PAD, D), jnp.float32),
    )(p, g, dinv, b, gamma, beta)


@jax.jit
def kernel(x, edge_idx, W, b, gamma, beta):
    src = edge_idx[0].astype(jnp.int32)
    dst = edge_idx[1].astype(jnp.int32)
    pad = jnp.full((E_PAD - E,), N, jnp.int32)
    src_ch = jnp.concatenate([src, pad]).reshape(N_CHUNK, 128)
    dst_ch = jnp.concatenate([dst, pad]).reshape(N_CHUNK, 128)

    deg = _deg_call(dst_ch.reshape(NW, CH, 128))
    g, dinv = _mm_call(x, W, deg)
    p = _agg_call(g, src_ch, dst_ch)
    y = _bn_call(p, g, dinv, b.reshape(1, D), gamma.reshape(1, D), beta.reshape(1, D))
    return y[:N]


# speedup vs baseline: 29.9642x; 1.9955x over previous
import functools

import jax
import jax.numpy as jnp
from jax import lax
from jax.experimental import pallas as pl
from jax.experimental.pallas import tpu as pltpu
from jax.experimental.pallas import tpu_sc as plsc

N = 10000
D = 128
E = 320000
ALPHA = 0.2

NC = 2
NS = 16
NW = NC * NS

CH = 80
E_PER_TILE = CH * 128
E_PAD = NW * E_PER_TILE
N_CHUNK = E_PAD // 128
CHT = N_CHUNK // NW
N_PAD = 10112
STRIPE = N_PAD // NS

_MESH = plsc.VectorSubcoreMesh(
    core_axis_name="c", subcore_axis_name="s", num_cores=NC, num_subcores=NS
)


DW = 16


def _deg_body(dst_hbm, deg_out, dst_v, ones_buf, acc):
    c = lax.axis_index("c")
    s = lax.axis_index("s")
    w = c * NS + s
    zeros16 = jnp.zeros((16,), jnp.float32)

    def zero(r, _):
        ones_buf[r, pl.ds(0, 16)] = zeros16
        return 0

    lax.fori_loop(0, 128, zero, 0)
    base = s * STRIPE
    for k in range(STRIPE // 128):
        pltpu.sync_copy(ones_buf, acc.at[pl.ds(base + k * 128, 128)])
    rem = STRIPE % 128
    pltpu.sync_copy(ones_buf.at[pl.ds(0, rem)], acc.at[pl.ds(base + STRIPE - rem, rem)])
    ones16 = jnp.ones((16,), jnp.float32)

    def fill(r, _):
        ones_buf[r, pl.ds(0, 16)] = ones16
        return 0

    lax.fori_loop(0, 128, fill, 0)
    pltpu.sync_copy(dst_hbm.at[w], dst_v)
    plsc.subcore_barrier()

    def step(j, _):
        pltpu.sync_copy(ones_buf, acc.at[dst_v.at[j]], add=True)
        return 0

    lax.fori_loop(0, CH, step, 0)
    plsc.subcore_barrier()
    pltpu.sync_copy(acc.at[pl.ds(base, STRIPE)], deg_out.at[c, pl.ds(base, STRIPE)])


_deg_call = pl.kernel(
    _deg_body,
    out_type=jax.ShapeDtypeStruct((NC, N_PAD, DW), jnp.float32),
    mesh=_MESH,
    scratch_types=[
        pltpu.VMEM((CH, 128), jnp.int32),
        pltpu.VMEM((128, DW), jnp.float32),
        pltpu.VMEM_SHARED((N_PAD, DW), jnp.float32),
    ],
)


MMB = N_PAD // 8


def _mm_body(x_ref, w_ref, deg_ref, g_ref, dinv_ref):
    i = pl.program_id(0)
    deg = (deg_ref[0] + deg_ref[1])[:, 0:1] + 1.0
    dv = lax.rsqrt(deg)
    row = i * MMB + lax.broadcasted_iota(jnp.int32, (MMB, 1), 0)
    dv = jnp.where(row < N, dv, 0.0)
    h = jnp.dot(x_ref[...], w_ref[...], preferred_element_type=jnp.float32)
    g_ref[...] = dv * h
    dinv_ref[...] = dv


def _mm_call(x, W, deg):
    return pl.pallas_call(
        _mm_body,
        grid=(N_PAD // MMB,),
        in_specs=[
            pl.BlockSpec((MMB, D), lambda i: (i, 0)),
            pl.BlockSpec((D, D), lambda i: (0, 0)),
            pl.BlockSpec((NC, MMB, DW), lambda i: (0, i, 0)),
        ],
        out_specs=[
            pl.BlockSpec((MMB, D), lambda i: (i, 0)),
            pl.BlockSpec((MMB, 1), lambda i: (i, 0)),
        ],
        out_shape=[
            jax.ShapeDtypeStruct((N_PAD, D), jnp.float32),
            jax.ShapeDtypeStruct((N_PAD, 1), jnp.float32),
        ],
    )(x, W, deg)


WIN = 8
NWINT = CHT // WIN


def _agg_body(g_hbm, src_hbm, dst_hbm, out_hbm,
              sw0, dw0, sw1, dw1, buf0, buf1, acc, sem_g, sem_g2, sem_w):
    c = lax.axis_index("c")
    s = lax.axis_index("s")
    zeros16 = jnp.zeros((16,), jnp.float32)

    def zero(r, _):
        for cc in range(8):
            buf0[r, pl.ds(cc * 16, 16)] = zeros16
        return 0

    lax.fori_loop(0, 128, zero, 0)
    base = s * STRIPE
    for k in range(STRIPE // 128):
        pltpu.sync_copy(buf0, acc.at[pl.ds(base + k * 128, 128)])
    rem = STRIPE % 128
    pltpu.sync_copy(buf0.at[pl.ds(0, rem)], acc.at[pl.ds(base + STRIPE - rem, rem)])

    start = pl.multiple_of((c * NS + s) * CHT, 8)
    pltpu.sync_copy(src_hbm.at[pl.ds(start, WIN)], sw0)
    pltpu.sync_copy(dst_hbm.at[pl.ds(start, WIN)], dw0)
    plsc.subcore_barrier()

    def window_body(wi, w_cur_s, w_cur_d, w_next_s, w_next_d):
        @pl.when(wi + 1 < NWINT)
        def _():
            off = pl.multiple_of(start + (wi + 1) * WIN, 8)
            pltpu.async_copy(src_hbm.at[pl.ds(off, WIN)], w_next_s, sem_w)
            pltpu.async_copy(dst_hbm.at[pl.ds(off, WIN)], w_next_d, sem_w)

        @pl.when(wi > 0)
        def _():
            pltpu.make_async_copy(src_hbm.at[pl.ds(0, WIN)], w_cur_s, sem_w).wait()
            pltpu.make_async_copy(dst_hbm.at[pl.ds(0, WIN)], w_cur_d, sem_w).wait()

        def pair(j2p, _):
            a = j2p * 2
            pltpu.async_copy(g_hbm.at[w_cur_s.at[a]], buf0, sem_g).wait()
            pltpu.sync_copy(buf0, acc.at[w_cur_d.at[a]], add=True)
            pltpu.async_copy(g_hbm.at[w_cur_s.at[a + 1]], buf1, sem_g2).wait()
            pltpu.sync_copy(buf1, acc.at[w_cur_d.at[a + 1]], add=True)
            return 0

        lax.fori_loop(0, WIN // 2, pair, 0)

    def outer(wi, _):
        @pl.when(lax.rem(wi, 2) == 0)
        def _():
            window_body(wi, sw0, dw0, sw1, dw1)

        @pl.when(lax.rem(wi, 2) == 1)
        def _():
            window_body(wi, sw1, dw1, sw0, dw0)

        return 0

    lax.fori_loop(0, NWINT, outer, 0)
    plsc.subcore_barrier()
    pltpu.sync_copy(acc.at[pl.ds(base, STRIPE)], out_hbm.at[c, pl.ds(base, STRIPE)])


_agg_call = pl.kernel(
    _agg_body,
    out_type=jax.ShapeDtypeStruct((NC, N_PAD, D), jnp.float32),
    mesh=_MESH,
    scratch_types=[
        pltpu.VMEM((WIN, 128), jnp.int32),
        pltpu.VMEM((WIN, 128), jnp.int32),
        pltpu.VMEM((WIN, 128), jnp.int32),
        pltpu.VMEM((WIN, 128), jnp.int32),
        pltpu.VMEM((128, D), jnp.float32),
        pltpu.VMEM((128, D), jnp.float32),
        pltpu.VMEM_SHARED((N_PAD, D), jnp.float32),
        pltpu.SemaphoreType.DMA,
        pltpu.SemaphoreType.DMA,
        pltpu.SemaphoreType.DMA,
    ],
)


def _bn_body(p_ref, g_ref, dinv_ref, b_ref, gamma_ref, beta_ref, y_ref):
    t = dinv_ref[...] * (p_ref[0] + p_ref[1] + g_ref[...]) + b_ref[...]
    row = lax.broadcasted_iota(jnp.int32, (N_PAD, 1), 0)
    tm = jnp.where(row < N, t, 0.0)
    s1 = jnp.sum(tm, axis=0, keepdims=True)
    s2 = jnp.sum(tm * tm, axis=0, keepdims=True)
    mean = s1 * (1.0 / N)
    var = s2 * (1.0 / N) - mean * mean
    inv = lax.rsqrt(var + 1e-5)
    y = (t - mean) * inv * gamma_ref[...] + beta_ref[...]
    y_ref[...] = jnp.where(y > 0, y, ALPHA * y)


def _bn_call(p, g, dinv, b, gamma, beta):
    return pl.pallas_call(
        _bn_body,
        out_shape=jax.ShapeDtypeStruct((N_PAD, D), jnp.float32),
    )(p, g, dinv, b, gamma, beta)


@jax.jit
def kernel(x, edge_idx, W, b, gamma, beta):
    src = edge_idx[0].astype(jnp.int32)
    dst = edge_idx[1].astype(jnp.int32)
    pad = N + jnp.arange(E_PAD - E, dtype=jnp.int32) % (N_PAD - N)
    src_ch = jnp.concatenate([src, pad]).reshape(N_CHUNK, 128)
    dst_ch = jnp.concatenate([dst, pad]).reshape(N_CHUNK, 128)

    deg = _deg_call(dst_ch.reshape(NW, CH, 128))
    g, dinv = _mm_call(x, W, deg)
    p = _agg_call(g, src_ch, dst_ch)
    y = _bn_call(p, g, dinv, b.reshape(1, D), gamma.reshape(1, D), beta.reshape(1, D))
    return y[:N]


# speedup vs baseline: 30.4244x; 1.0154x over previous
import functools

import jax
import jax.numpy as jnp
from jax import lax
from jax.experimental import pallas as pl
from jax.experimental.pallas import tpu as pltpu
from jax.experimental.pallas import tpu_sc as plsc

N = 10000
D = 128
E = 320000
ALPHA = 0.2

NC = 2
NS = 16
NW = NC * NS

CH = 80
E_PER_TILE = CH * 128
E_PAD = NW * E_PER_TILE
N_CHUNK = E_PAD // 128
CHT = N_CHUNK // NW
N_PAD = 10112
STRIPE = N_PAD // NS

_MESH = plsc.VectorSubcoreMesh(
    core_axis_name="c", subcore_axis_name="s", num_cores=NC, num_subcores=NS
)


DW = 16


def _deg_body(dst_hbm, deg_out, dst_v, ones_buf, acc):
    c = lax.axis_index("c")
    s = lax.axis_index("s")
    w = c * NS + s
    zeros16 = jnp.zeros((16,), jnp.float32)

    def zero(r, _):
        ones_buf[r, pl.ds(0, 16)] = zeros16
        return 0

    lax.fori_loop(0, 128, zero, 0)
    base = s * STRIPE
    for k in range(STRIPE // 128):
        pltpu.sync_copy(ones_buf, acc.at[pl.ds(base + k * 128, 128)])
    rem = STRIPE % 128
    pltpu.sync_copy(ones_buf.at[pl.ds(0, rem)], acc.at[pl.ds(base + STRIPE - rem, rem)])
    ones16 = jnp.ones((16,), jnp.float32)

    def fill(r, _):
        ones_buf[r, pl.ds(0, 16)] = ones16
        return 0

    lax.fori_loop(0, 128, fill, 0)
    pltpu.sync_copy(dst_hbm.at[w], dst_v)
    plsc.subcore_barrier()

    def step(j, _):
        pltpu.sync_copy(ones_buf, acc.at[dst_v.at[j]], add=True)
        return 0

    lax.fori_loop(0, CH, step, 0)
    plsc.subcore_barrier()
    pltpu.sync_copy(acc.at[pl.ds(base, STRIPE)], deg_out.at[c, pl.ds(base, STRIPE)])


_deg_call = pl.kernel(
    _deg_body,
    out_type=jax.ShapeDtypeStruct((NC, N_PAD, DW), jnp.float32),
    mesh=_MESH,
    scratch_types=[
        pltpu.VMEM((CH, 128), jnp.int32),
        pltpu.VMEM((128, DW), jnp.float32),
        pltpu.VMEM_SHARED((N_PAD, DW), jnp.float32),
    ],
)


MMB = N_PAD // 8


def _mm_body(x_ref, w_ref, deg_ref, g_ref, dinv_ref):
    i = pl.program_id(0)
    deg = (deg_ref[0] + deg_ref[1])[:, 0:1] + 1.0
    dv = lax.rsqrt(deg)
    row = i * MMB + lax.broadcasted_iota(jnp.int32, (MMB, 1), 0)
    dv = jnp.where(row < N, dv, 0.0)
    h = jnp.dot(x_ref[...], w_ref[...], preferred_element_type=jnp.float32)
    g_ref[...] = dv * h
    dinv_ref[...] = dv


def _mm_call(x, W, deg):
    return pl.pallas_call(
        _mm_body,
        grid=(N_PAD // MMB,),
        in_specs=[
            pl.BlockSpec((MMB, D), lambda i: (i, 0)),
            pl.BlockSpec((D, D), lambda i: (0, 0)),
            pl.BlockSpec((NC, MMB, DW), lambda i: (0, i, 0)),
        ],
        out_specs=[
            pl.BlockSpec((MMB, D), lambda i: (i, 0)),
            pl.BlockSpec((MMB, 1), lambda i: (i, 0)),
        ],
        out_shape=[
            jax.ShapeDtypeStruct((N_PAD, D), jnp.float32),
            jax.ShapeDtypeStruct((N_PAD, 1), jnp.float32),
        ],
    )(x, W, deg)


def _agg_body(g_hbm, src_hbm, dst_hbm, out_hbm, src_v, dst_v, buf, acc, sem_g):
    c = lax.axis_index("c")
    s = lax.axis_index("s")
    zeros16 = jnp.zeros((16,), jnp.float32)

    def zero(r, _):
        for cc in range(8):
            buf[r, pl.ds(cc * 16, 16)] = zeros16
        return 0

    lax.fori_loop(0, 128, zero, 0)
    base = s * STRIPE
    for k in range(STRIPE // 128):
        pltpu.sync_copy(buf, acc.at[pl.ds(base + k * 128, 128)])
    rem = STRIPE % 128
    pltpu.sync_copy(buf.at[pl.ds(0, rem)], acc.at[pl.ds(base + STRIPE - rem, rem)])

    start = pl.multiple_of((c * NS + s) * CHT, 8)
    pltpu.sync_copy(src_hbm.at[pl.ds(start, CHT)], src_v)
    pltpu.sync_copy(dst_hbm.at[pl.ds(start, CHT)], dst_v)
    plsc.subcore_barrier()

    def step(j, _):
        pltpu.async_copy(g_hbm.at[src_v.at[j]], buf, sem_g).wait()
        pltpu.sync_copy(buf, acc.at[dst_v.at[j]], add=True)
        return 0

    lax.fori_loop(0, CHT, step, 0)
    plsc.subcore_barrier()
    pltpu.sync_copy(acc.at[pl.ds(base, STRIPE)], out_hbm.at[c, pl.ds(base, STRIPE)])
    plsc.subcore_barrier()


_agg_call = pl.kernel(
    _agg_body,
    out_type=jax.ShapeDtypeStruct((NC, N_PAD, D), jnp.float32),
    mesh=_MESH,
    scratch_types=[
        pltpu.VMEM((CHT, 128), jnp.int32),
        pltpu.VMEM((CHT, 128), jnp.int32),
        pltpu.VMEM((128, D), jnp.float32),
        pltpu.VMEM_SHARED((N_PAD, D), jnp.float32),
        pltpu.SemaphoreType.DMA,
    ],
)


def _bn_body(p_ref, g_ref, dinv_ref, b_ref, gamma_ref, beta_ref, y_ref):
    t = dinv_ref[...] * (p_ref[0] + p_ref[1] + g_ref[...]) + b_ref[...]
    row = lax.broadcasted_iota(jnp.int32, (N_PAD, 1), 0)
    tm = jnp.where(row < N, t, 0.0)
    s1 = jnp.sum(tm, axis=0, keepdims=True)
    s2 = jnp.sum(tm * tm, axis=0, keepdims=True)
    mean = s1 * (1.0 / N)
    var = s2 * (1.0 / N) - mean * mean
    inv = lax.rsqrt(var + 1e-5)
    y = (t[:N] - mean) * inv * gamma_ref[...] + beta_ref[...]
    y_ref[...] = jnp.where(y > 0, y, ALPHA * y)


def _bn_call(p, g, dinv, b, gamma, beta):
    return pl.pallas_call(
        _bn_body,
        out_shape=jax.ShapeDtypeStruct((N, D), jnp.float32),
    )(p, g, dinv, b, gamma, beta)


@jax.jit
def kernel(x, edge_idx, W, b, gamma, beta):
    src = edge_idx[0].astype(jnp.int32)
    dst = edge_idx[1].astype(jnp.int32)
    pad = N + jnp.arange(E_PAD - E, dtype=jnp.int32) % (N_PAD - N)
    src_ch = jnp.concatenate([src, pad]).reshape(N_CHUNK, 128)
    dst_ch = jnp.concatenate([dst, pad]).reshape(N_CHUNK, 128)

    deg = _deg_call(dst_ch.reshape(NW, CH, 128))
    g, dinv = _mm_call(x, W, deg)
    p = _agg_call(g, src_ch, dst_ch)
    return _bn_call(p, g, dinv, b.reshape(1, D), gamma.reshape(1, D), beta.reshape(1, D))
